# baseline (speedup 1.0000x reference)
import functools

import jax
import jax.numpy as jnp
from jax import lax
from jax.experimental import pallas as pl
from jax.experimental.pallas import tpu as pltpu
from jax.experimental.pallas import tpu_sc as plsc

_BATCH = 16384
_HIDDEN = 64
_NUM_CORES = 2
_NUM_SUBCORES = 16
_NW = _NUM_CORES * _NUM_SUBCORES
_BPW = _BATCH // _NW

_sc_mesh = plsc.VectorSubcoreMesh(core_axis_name="c", subcore_axis_name="s")


@functools.partial(
    pl.kernel,
    out_type=(
        jax.ShapeDtypeStruct((_BATCH, _HIDDEN), jnp.float32),
        jax.ShapeDtypeStruct((_BATCH, _HIDDEN), jnp.float32),
    ),
    mesh=_sc_mesh,
    compiler_params=pltpu.CompilerParams(use_tc_tiling_on_sc=False),
    scratch_types=[
        pltpu.VMEM((_BPW,), jnp.int32),
        pltpu.VMEM((_BPW,), jnp.int32),
        pltpu.VMEM((_BPW, _HIDDEN), jnp.float32),
        pltpu.VMEM((_BPW, _HIDDEN), jnp.float32),
        pltpu.SemaphoreType.DMA,
        pltpu.SemaphoreType.DMA,
    ],
)
def _sc_gather(uid_hbm, iid_hbm, utab_hbm, itab_hbm, uout_hbm, iout_hbm,
               uidx_v, iidx_v, urows_v, irows_v, usem, isem):
    wid = lax.axis_index("s") * _NUM_CORES + lax.axis_index("c")
    base = wid * _BPW
    pltpu.sync_copy(uid_hbm.at[pl.ds(base, _BPW)], uidx_v)
    pltpu.sync_copy(iid_hbm.at[pl.ds(base, _BPW)], iidx_v)
    ucp = pltpu.async_copy(utab_hbm.at[uidx_v], urows_v, usem)
    icp = pltpu.async_copy(itab_hbm.at[iidx_v], irows_v, isem)
    ucp.wait()
    pltpu.sync_copy(urows_v, uout_hbm.at[pl.ds(base, _BPW)])
    icp.wait()
    pltpu.sync_copy(irows_v, iout_hbm.at[pl.ds(base, _BPW)])


_ROWS_BLK = 2048


def _norm_body(u_ref, i_ref, uo_ref, io_ref):
    for src, dst in ((u_ref, uo_ref), (i_ref, io_ref)):
        x = src[...]
        norm = jnp.sqrt(jnp.sum(x * x, axis=1, keepdims=True))
        dst[...] = x / jnp.maximum(norm, 1e-12)


_tc_normalize = pl.pallas_call(
    _norm_body,
    grid=(_BATCH // _ROWS_BLK,),
    in_specs=[pl.BlockSpec((_ROWS_BLK, _HIDDEN), lambda i: (i, 0))] * 2,
    out_specs=[pl.BlockSpec((_ROWS_BLK, _HIDDEN), lambda i: (i, 0))] * 2,
    out_shape=(
        jax.ShapeDtypeStruct((_BATCH, _HIDDEN), jnp.float32),
        jax.ShapeDtypeStruct((_BATCH, _HIDDEN), jnp.float32),
    ),
)


def kernel(user_ids, item_ids, user_table, item_table):
    uid = user_ids.astype(jnp.int32)
    iid = item_ids.astype(jnp.int32)
    uraw, iraw = _sc_gather(uid, iid, user_table, item_table)
    return _tc_normalize(uraw, iraw)
